# Initial kernel scaffold; baseline (speedup 1.0000x reference)
import jax
import jax.numpy as jnp
from jax.experimental import pallas as pl


def kernel(x, edge_index, W, b):
    raise NotImplementedError("write your pallas kernel here")


# speedup vs baseline: 15.6437x; 15.6437x over previous
import functools

import jax
import jax.numpy as jnp
from jax import lax
from jax.experimental import pallas as pl
from jax.experimental.pallas import tpu as pltpu
from jax.experimental.pallas import tpu_sc as plsc

N = 10000
NP = 10240
D = 128
NC = 2
NS = 16
NW = NC * NS
CHUNK = 128
RPT = NP // NS
HW = 16


def _mesh():
  return plsc.VectorSubcoreMesh(core_axis_name="c", subcore_axis_name="s")


def _sc_degree(col3, zeros_h, ones_h, nchunk):
  @functools.partial(
      pl.kernel,
      out_type=jax.ShapeDtypeStruct((NC, NP), jnp.float32),
      mesh=_mesh(),
      scratch_types=[
          pltpu.VMEM((CHUNK,), jnp.int32),
          pltpu.VMEM((CHUNK,), jnp.float32),
          pltpu.VMEM_SHARED((NP,), jnp.float32),
      ],
  )
  def k(col_hbm, zeros_hbm, ones_hbm, out_hbm, col_c, ones_v, hacc):
    c = lax.axis_index("c")
    s = lax.axis_index("s")
    wid = c * NS + s
    pltpu.sync_copy(ones_hbm, ones_v)
    zr = NP // NS
    pltpu.sync_copy(zeros_hbm.at[pl.ds(s * zr, zr)], hacc.at[pl.ds(s * zr, zr)])
    plsc.subcore_barrier()

    def body(j, carry):
      pltpu.sync_copy(col_hbm.at[wid, j], col_c)
      pltpu.sync_copy(ones_v, hacc.at[col_c], add=True)
      return carry

    lax.fori_loop(0, nchunk, body, 0)
    plsc.subcore_barrier()
    pltpu.sync_copy(hacc.at[pl.ds(s * zr, zr)], out_hbm.at[c, pl.ds(s * zr, zr)])

  return k(col3, zeros_h, ones_h)


def _sc_scatter(y, row3, col3, nchunk):
  @functools.partial(
      pl.kernel,
      out_type=jax.ShapeDtypeStruct((NC, NP, D), jnp.float32),
      mesh=_mesh(),
      scratch_types=[
          pltpu.VMEM((CHUNK,), jnp.int32),
          pltpu.VMEM((CHUNK,), jnp.int32),
          pltpu.VMEM((CHUNK, D), jnp.float32),
          pltpu.VMEM_SHARED((NP, D), jnp.float32),
      ],
  )
  def k(y_hbm, row_hbm, col_hbm, out_hbm, row_c, col_c, msg, accum):
    c = lax.axis_index("c")
    s = lax.axis_index("s")
    wid = c * NS + s
    r0 = s * RPT
    pltpu.sync_copy(y_hbm.at[pl.ds(r0, RPT)], accum.at[pl.ds(r0, RPT)])
    plsc.subcore_barrier()

    def body(j, carry):
      pltpu.sync_copy(row_hbm.at[wid, j], row_c)
      pltpu.sync_copy(col_hbm.at[wid, j], col_c)
      pltpu.sync_copy(y_hbm.at[row_c], msg)
      pltpu.sync_copy(msg, accum.at[col_c], add=True)
      return carry

    lax.fori_loop(0, nchunk, body, 0)
    plsc.subcore_barrier()
    pltpu.sync_copy(accum.at[pl.ds(r0, RPT)], out_hbm.at[c, pl.ds(r0, RPT)])

  return k(y, row3, col3)


BLK = 1024


def _tc_first(x, W, h0, h1):
  def body(x_ref, w_ref, h0_ref, h1_ref, y_ref, dis_ref):
    deg = h0_ref[...] + h1_ref[...] + 1.0
    dis = lax.rsqrt(deg)
    xw = jnp.dot(x_ref[...], w_ref[...], preferred_element_type=jnp.float32)
    y_ref[...] = xw * dis
    dis_ref[...] = dis

  return pl.pallas_call(
      body,
      grid=(NP // BLK,),
      in_specs=[
          pl.BlockSpec((BLK, D), lambda i: (i, 0)),
          pl.BlockSpec((D, D), lambda i: (0, 0)),
          pl.BlockSpec((BLK, 1), lambda i: (i, 0)),
          pl.BlockSpec((BLK, 1), lambda i: (i, 0)),
      ],
      out_specs=[
          pl.BlockSpec((BLK, D), lambda i: (i, 0)),
          pl.BlockSpec((BLK, 1), lambda i: (i, 0)),
      ],
      out_shape=[
          jax.ShapeDtypeStruct((NP, D), jnp.float32),
          jax.ShapeDtypeStruct((NP, 1), jnp.float32),
      ],
  )(x, W, h0, h1)


def _tc_mid(p, y, dis, b2, W):
  def body(p0_ref, p1_ref, y_ref, dis_ref, b_ref, w_ref, out_ref):
    d = dis_ref[...]
    t = jnp.maximum(
        d * (p0_ref[...] + p1_ref[...] - y_ref[...]) + b_ref[...], 0.0
    )
    out_ref[...] = d * jnp.dot(
        t, w_ref[...], preferred_element_type=jnp.float32
    )

  return pl.pallas_call(
      body,
      grid=(NP // BLK,),
      in_specs=[
          pl.BlockSpec((BLK, D), lambda i: (i, 0)),
          pl.BlockSpec((BLK, D), lambda i: (i, 0)),
          pl.BlockSpec((BLK, D), lambda i: (i, 0)),
          pl.BlockSpec((BLK, 1), lambda i: (i, 0)),
          pl.BlockSpec((1, D), lambda i: (0, 0)),
          pl.BlockSpec((D, D), lambda i: (0, 0)),
      ],
      out_specs=pl.BlockSpec((BLK, D), lambda i: (i, 0)),
      out_shape=jax.ShapeDtypeStruct((NP, D), jnp.float32),
  )(p[0], p[1], y, dis, b2, W)


def _tc_last(q, y, dis, b2):
  def body(q0_ref, q1_ref, y_ref, dis_ref, b_ref, out_ref):
    d = dis_ref[...]
    out_ref[...] = jnp.maximum(
        d * (q0_ref[...] + q1_ref[...] - y_ref[...]) + b_ref[...], 0.0
    )

  return pl.pallas_call(
      body,
      grid=(NP // BLK,),
      in_specs=[
          pl.BlockSpec((BLK, D), lambda i: (i, 0)),
          pl.BlockSpec((BLK, D), lambda i: (i, 0)),
          pl.BlockSpec((BLK, D), lambda i: (i, 0)),
          pl.BlockSpec((BLK, 1), lambda i: (i, 0)),
          pl.BlockSpec((1, D), lambda i: (0, 0)),
      ],
      out_specs=pl.BlockSpec((BLK, D), lambda i: (i, 0)),
      out_shape=jax.ShapeDtypeStruct((NP, D), jnp.float32),
  )(q[0], q[1], y, dis, b2)


def kernel(x, edge_index, W, b):
  E = edge_index.shape[1]
  row = edge_index[0].astype(jnp.int32)
  col = edge_index[1].astype(jnp.int32)

  epw = -(-E // NW)
  nchunk = -(-epw // CHUNK)
  pad = nchunk * CHUNK * NW - E
  pad_ar = jnp.arange(pad, dtype=jnp.int32)
  row3 = jnp.concatenate([row, pad_ar % N]).reshape(NW, nchunk, CHUNK)
  col3 = jnp.concatenate([col, N + pad_ar % (NP - N)]).reshape(
      NW, nchunk, CHUNK)
  b2 = b.reshape(1, D)
  zeros_h = jnp.zeros((NP,), jnp.float32)
  ones_h = jnp.ones((CHUNK,), jnp.float32)
  xp = jnp.pad(x, ((0, NP - N), (0, 0)))

  h = _sc_degree(col3, zeros_h, ones_h, nchunk)
  y, dis = _tc_first(xp, W, h[0].reshape(NP, 1), h[1].reshape(NP, 1))
  p = _sc_scatter(y, row3, col3, nchunk)
  y2 = _tc_mid(p, y, dis, b2, W)
  q = _sc_scatter(y2, row3, col3, nchunk)
  return _tc_last(q, y2, dis, b2)[:N]


# speedup vs baseline: 24.8683x; 1.5897x over previous
import functools

import jax
import jax.numpy as jnp
from jax import lax
from jax.experimental import pallas as pl
from jax.experimental.pallas import tpu as pltpu
from jax.experimental.pallas import tpu_sc as plsc

N = 10000
NP = 10240
D = 128
NC = 2
NS = 16
NW = NC * NS
CHUNK = 128
RPT = NP // NS
PH = 40
HW = 16


def _mesh():
  return plsc.VectorSubcoreMesh(core_axis_name="c", subcore_axis_name="s")


def _sc_degree(col3, zeros_h, ones_h, nchunk):
  @functools.partial(
      pl.kernel,
      out_type=jax.ShapeDtypeStruct((NC, NP), jnp.float32),
      mesh=_mesh(),
      scratch_types=[
          pltpu.VMEM((nchunk, CHUNK), jnp.int32),
          pltpu.VMEM((CHUNK,), jnp.float32),
          pltpu.VMEM_SHARED((NP,), jnp.float32),
          pltpu.SemaphoreType.DMA,
      ],
  )
  def k(col_hbm, zeros_hbm, ones_hbm, out_hbm, col_v, ones_v, hacc, sem):
    c = lax.axis_index("c")
    s = lax.axis_index("s")
    wid = c * NS + s
    pltpu.sync_copy(ones_hbm, ones_v)
    pltpu.sync_copy(col_hbm.at[wid], col_v)
    zr = NP // NS
    pltpu.sync_copy(zeros_hbm.at[pl.ds(s * zr, zr)], hacc.at[pl.ds(s * zr, zr)])
    plsc.subcore_barrier()

    def body(j, carry):
      pltpu.async_copy(ones_v, hacc.at[col_v.at[j]], sem, add=True)
      return carry

    lax.fori_loop(0, nchunk, body, 0)

    def drain(j, carry):
      pltpu.make_async_copy(ones_v, hacc.at[col_v.at[0]], sem).wait()
      return carry

    lax.fori_loop(0, nchunk, drain, 0)
    plsc.subcore_barrier()
    pltpu.sync_copy(hacc.at[pl.ds(s * zr, zr)], out_hbm.at[c, pl.ds(s * zr, zr)])

  return k(col3, zeros_h, ones_h)


def _sc_scatter(y, row3, col3, nchunk):
  nph = nchunk // PH

  @functools.partial(
      pl.kernel,
      out_type=jax.ShapeDtypeStruct((NC, NP, D), jnp.float32),
      mesh=_mesh(),
      scratch_types=[
          pltpu.VMEM((PH, CHUNK), jnp.int32),
          pltpu.VMEM((PH, CHUNK), jnp.int32),
          pltpu.VMEM((CHUNK, D), jnp.float32),
          pltpu.VMEM((CHUNK, D), jnp.float32),
          pltpu.VMEM_SHARED((NP, D), jnp.float32),
          pltpu.SemaphoreType.DMA,
          pltpu.SemaphoreType.DMA,
          pltpu.SemaphoreType.DMA,
          pltpu.SemaphoreType.DMA,
      ],
  )
  def k(y_hbm, row_hbm, col_hbm, out_hbm, row_v, col_v, msg0, msg1, accum,
        semg0, semg1, sems0, sems1):
    c = lax.axis_index("c")
    s = lax.axis_index("s")
    wid = c * NS + s
    r0 = s * RPT
    pltpu.sync_copy(y_hbm.at[pl.ds(r0, RPT)], accum.at[pl.ds(r0, RPT)])
    plsc.subcore_barrier()

    for ph in range(nph):
      pltpu.sync_copy(row_hbm.at[wid, pl.ds(ph * PH, PH)], row_v)
      pltpu.sync_copy(col_hbm.at[wid, pl.ds(ph * PH, PH)], col_v)
      pltpu.async_copy(y_hbm.at[row_v.at[0]], msg0, semg0)
      pltpu.async_copy(y_hbm.at[row_v.at[1]], msg1, semg1)

      def body(i, carry):
        j0 = 2 * i
        pltpu.make_async_copy(y_hbm.at[row_v.at[j0]], msg0, semg0).wait()
        pltpu.async_copy(msg0, accum.at[col_v.at[j0]], sems0, add=True)
        pltpu.make_async_copy(y_hbm.at[row_v.at[j0 + 1]], msg1, semg1).wait()
        pltpu.async_copy(msg1, accum.at[col_v.at[j0 + 1]], sems1, add=True)

        @pl.when(j0 + 2 < PH)
        def _():
          pltpu.make_async_copy(msg0, accum.at[col_v.at[0]], sems0).wait()
          pltpu.async_copy(y_hbm.at[row_v.at[j0 + 2]], msg0, semg0)

        @pl.when(j0 + 3 < PH)
        def _():
          pltpu.make_async_copy(msg1, accum.at[col_v.at[0]], sems1).wait()
          pltpu.async_copy(y_hbm.at[row_v.at[j0 + 3]], msg1, semg1)

        return carry

      lax.fori_loop(0, PH // 2, body, 0)
      pltpu.make_async_copy(msg0, accum.at[col_v.at[0]], sems0).wait()
      pltpu.make_async_copy(msg1, accum.at[col_v.at[0]], sems1).wait()

    plsc.subcore_barrier()
    pltpu.sync_copy(accum.at[pl.ds(r0, RPT)], out_hbm.at[c, pl.ds(r0, RPT)])

  return k(y, row3, col3)


BLK = 1024


def _tc_first(x, W, h0, h1):
  def body(x_ref, w_ref, h0_ref, h1_ref, y_ref, dis_ref):
    deg = h0_ref[...] + h1_ref[...] + 1.0
    dis = lax.rsqrt(deg)
    xw = jnp.dot(x_ref[...], w_ref[...], preferred_element_type=jnp.float32)
    y_ref[...] = xw * dis
    dis_ref[...] = dis

  return pl.pallas_call(
      body,
      grid=(NP // BLK,),
      in_specs=[
          pl.BlockSpec((BLK, D), lambda i: (i, 0)),
          pl.BlockSpec((D, D), lambda i: (0, 0)),
          pl.BlockSpec((BLK, 1), lambda i: (i, 0)),
          pl.BlockSpec((BLK, 1), lambda i: (i, 0)),
      ],
      out_specs=[
          pl.BlockSpec((BLK, D), lambda i: (i, 0)),
          pl.BlockSpec((BLK, 1), lambda i: (i, 0)),
      ],
      out_shape=[
          jax.ShapeDtypeStruct((NP, D), jnp.float32),
          jax.ShapeDtypeStruct((NP, 1), jnp.float32),
      ],
  )(x, W, h0, h1)


def _tc_mid(p, y, dis, b2, W):
  def body(p0_ref, p1_ref, y_ref, dis_ref, b_ref, w_ref, out_ref):
    d = dis_ref[...]
    t = jnp.maximum(
        d * (p0_ref[...] + p1_ref[...] - y_ref[...]) + b_ref[...], 0.0
    )
    out_ref[...] = d * jnp.dot(
        t, w_ref[...], preferred_element_type=jnp.float32
    )

  return pl.pallas_call(
      body,
      grid=(NP // BLK,),
      in_specs=[
          pl.BlockSpec((BLK, D), lambda i: (i, 0)),
          pl.BlockSpec((BLK, D), lambda i: (i, 0)),
          pl.BlockSpec((BLK, D), lambda i: (i, 0)),
          pl.BlockSpec((BLK, 1), lambda i: (i, 0)),
          pl.BlockSpec((1, D), lambda i: (0, 0)),
          pl.BlockSpec((D, D), lambda i: (0, 0)),
      ],
      out_specs=pl.BlockSpec((BLK, D), lambda i: (i, 0)),
      out_shape=jax.ShapeDtypeStruct((NP, D), jnp.float32),
  )(p[0], p[1], y, dis, b2, W)


def _tc_last(q, y, dis, b2):
  def body(q0_ref, q1_ref, y_ref, dis_ref, b_ref, out_ref):
    d = dis_ref[...]
    out_ref[...] = jnp.maximum(
        d * (q0_ref[...] + q1_ref[...] - y_ref[...]) + b_ref[...], 0.0
    )

  return pl.pallas_call(
      body,
      grid=(NP // BLK,),
      in_specs=[
          pl.BlockSpec((BLK, D), lambda i: (i, 0)),
          pl.BlockSpec((BLK, D), lambda i: (i, 0)),
          pl.BlockSpec((BLK, D), lambda i: (i, 0)),
          pl.BlockSpec((BLK, 1), lambda i: (i, 0)),
          pl.BlockSpec((1, D), lambda i: (0, 0)),
      ],
      out_specs=pl.BlockSpec((BLK, D), lambda i: (i, 0)),
      out_shape=jax.ShapeDtypeStruct((NP, D), jnp.float32),
  )(q[0], q[1], y, dis, b2)


def kernel(x, edge_index, W, b):
  E = edge_index.shape[1]
  row = edge_index[0].astype(jnp.int32)
  col = edge_index[1].astype(jnp.int32)

  epw = -(-E // NW)
  nchunk = -(-epw // CHUNK)
  if nchunk % PH:
    nchunk = (nchunk // PH + 1) * PH
  pad = nchunk * CHUNK * NW - E
  pad_ar = jnp.arange(pad, dtype=jnp.int32)
  row3 = jnp.concatenate([row, pad_ar % N]).reshape(NW, nchunk, CHUNK)
  col3 = jnp.concatenate([col, N + pad_ar % (NP - N)]).reshape(
      NW, nchunk, CHUNK)
  b2 = b.reshape(1, D)
  zeros_h = jnp.zeros((NP,), jnp.float32)
  ones_h = jnp.ones((CHUNK,), jnp.float32)
  xp = jnp.pad(x, ((0, NP - N), (0, 0)))

  h = _sc_degree(col3, zeros_h, ones_h, nchunk)
  y, dis = _tc_first(xp, W, h[0].reshape(NP, 1), h[1].reshape(NP, 1))
  p = _sc_scatter(y, row3, col3, nchunk)
  y2 = _tc_mid(p, y, dis, b2, W)
  q = _sc_scatter(y2, row3, col3, nchunk)
  return _tc_last(q, y2, dis, b2)[:N]


# speedup vs baseline: 27.6579x; 1.1122x over previous
import functools

import jax
import jax.numpy as jnp
from jax import lax
from jax.experimental import pallas as pl
from jax.experimental.pallas import tpu as pltpu
from jax.experimental.pallas import tpu_sc as plsc

N = 10000
NP = 10240
D = 128
NC = 2
NS = 16
NW = NC * NS
CHUNK = 128
RPT = NP // NS
SCH = 64
SPH = 40
DPH = 128
HW = 16


def _mesh():
  return plsc.VectorSubcoreMesh(core_axis_name="c", subcore_axis_name="s")


def _sc_degree(col3, zeros_h, ones_h, nchunk):
  @functools.partial(
      pl.kernel,
      out_type=jax.ShapeDtypeStruct((NC, NP), jnp.float32),
      mesh=_mesh(),
      scratch_types=[
          pltpu.VMEM((nchunk, DPH), jnp.int32),
          pltpu.VMEM((DPH,), jnp.float32),
          pltpu.VMEM_SHARED((NP,), jnp.float32),
          pltpu.SemaphoreType.DMA,
      ],
  )
  def k(col_hbm, zeros_hbm, ones_hbm, out_hbm, col_v, ones_v, hacc, sem):
    c = lax.axis_index("c")
    s = lax.axis_index("s")
    wid = c * NS + s
    pltpu.sync_copy(ones_hbm, ones_v)
    pltpu.sync_copy(col_hbm.at[wid], col_v)
    zr = NP // NS
    pltpu.sync_copy(zeros_hbm.at[pl.ds(s * zr, zr)], hacc.at[pl.ds(s * zr, zr)])
    plsc.subcore_barrier()

    def body(j, carry):
      pltpu.async_copy(ones_v, hacc.at[col_v.at[j]], sem, add=True)
      return carry

    lax.fori_loop(0, nchunk, body, 0)

    def drain(j, carry):
      pltpu.make_async_copy(ones_v, hacc.at[col_v.at[0]], sem).wait()
      return carry

    lax.fori_loop(0, nchunk, drain, 0)
    plsc.subcore_barrier()
    pltpu.sync_copy(hacc.at[pl.ds(s * zr, zr)], out_hbm.at[c, pl.ds(s * zr, zr)])

  return k(col3, zeros_h, ones_h)


def _sc_scatter(y, row4, col4, nchunk):
  nph = nchunk // SPH

  @functools.partial(
      pl.kernel,
      out_type=jax.ShapeDtypeStruct((NC, NP, D), jnp.float32),
      mesh=_mesh(),
      scratch_types=[
          pltpu.VMEM((SPH, SCH), jnp.int32),
          pltpu.VMEM((SPH, SCH), jnp.int32),
          pltpu.VMEM((SCH, D), jnp.float32),
          pltpu.VMEM((SCH, D), jnp.float32),
          pltpu.VMEM((SCH, D), jnp.float32),
          pltpu.VMEM((SCH, D), jnp.float32),
          pltpu.SemaphoreType.DMA,
          pltpu.SemaphoreType.DMA,
          pltpu.SemaphoreType.DMA,
          pltpu.SemaphoreType.DMA,
          pltpu.SemaphoreType.DMA,
          pltpu.SemaphoreType.DMA,
          pltpu.SemaphoreType.DMA,
          pltpu.SemaphoreType.DMA,
          pltpu.VMEM_SHARED((NP, D), jnp.float32),
      ],
  )
  def k(y_hbm, row_hbm, col_hbm, out_hbm, row_v, col_v, m0, m1, m2, m3,
        g0, g1, g2, g3, s0, s1, s2, s3, accum):
    c = lax.axis_index("c")
    s = lax.axis_index("s")
    wid = c * NS + s
    r0 = s * RPT
    msgs = (m0, m1, m2, m3)
    gsems = (g0, g1, g2, g3)
    ssems = (s0, s1, s2, s3)
    pltpu.sync_copy(y_hbm.at[pl.ds(r0, RPT)], accum.at[pl.ds(r0, RPT)])
    plsc.subcore_barrier()

    pltpu.sync_copy(row_hbm.at[wid, 0], row_v)
    pltpu.sync_copy(col_hbm.at[wid, 0], col_v)
    pltpu.async_copy(y_hbm.at[row_v.at[0]], msgs[0], gsems[0])
    pltpu.async_copy(y_hbm.at[row_v.at[1]], msgs[1], gsems[1])

    for ph in range(nph):

      def body(i, carry):
        for bi in range(4):
          j = 4 * i + bi
          b2 = (bi + 2) % 4
          pltpu.make_async_copy(y_hbm.at[row_v.at[j]], msgs[bi], gsems[bi]).wait()
          pltpu.async_copy(msgs[bi], accum.at[col_v.at[j]], ssems[bi], add=True)

          @pl.when(j + 2 < SPH)
          def _():
            if ph == 0:
              @pl.when(j >= 2)
              def _():
                pltpu.make_async_copy(msgs[b2], accum.at[col_v.at[0]],
                                      ssems[b2]).wait()
            else:
              pltpu.make_async_copy(msgs[b2], accum.at[col_v.at[0]],
                                    ssems[b2]).wait()
            pltpu.async_copy(y_hbm.at[row_v.at[j + 2]], msgs[b2], gsems[b2])

        return carry

      lax.fori_loop(0, SPH // 4, body, 0)
      if ph + 1 < nph:
        pltpu.make_async_copy(msgs[0], accum.at[col_v.at[0]], ssems[0]).wait()
        pltpu.make_async_copy(msgs[1], accum.at[col_v.at[0]], ssems[1]).wait()
        pltpu.sync_copy(row_hbm.at[wid, ph + 1], row_v)
        pltpu.sync_copy(col_hbm.at[wid, ph + 1], col_v)
        pltpu.async_copy(y_hbm.at[row_v.at[0]], msgs[0], gsems[0])
        pltpu.async_copy(y_hbm.at[row_v.at[1]], msgs[1], gsems[1])

    for b in range(4):
      pltpu.make_async_copy(msgs[b], accum.at[col_v.at[0]], ssems[b]).wait()

    plsc.subcore_barrier()
    pltpu.sync_copy(accum.at[pl.ds(r0, RPT)], out_hbm.at[c, pl.ds(r0, RPT)])

  return k(y, row4, col4)


BLK = 1024


def _tc_first(x, W, h0, h1):
  def body(x_ref, w_ref, h0_ref, h1_ref, y_ref, dis_ref):
    deg = h0_ref[...] + h1_ref[...] + 1.0
    dis = lax.rsqrt(deg)
    xw = jnp.dot(x_ref[...], w_ref[...], preferred_element_type=jnp.float32)
    y_ref[...] = xw * dis
    dis_ref[...] = dis

  return pl.pallas_call(
      body,
      grid=(NP // BLK,),
      in_specs=[
          pl.BlockSpec((BLK, D), lambda i: (i, 0)),
          pl.BlockSpec((D, D), lambda i: (0, 0)),
          pl.BlockSpec((BLK, 1), lambda i: (i, 0)),
          pl.BlockSpec((BLK, 1), lambda i: (i, 0)),
      ],
      out_specs=[
          pl.BlockSpec((BLK, D), lambda i: (i, 0)),
          pl.BlockSpec((BLK, 1), lambda i: (i, 0)),
      ],
      out_shape=[
          jax.ShapeDtypeStruct((NP, D), jnp.float32),
          jax.ShapeDtypeStruct((NP, 1), jnp.float32),
      ],
  )(x, W, h0, h1)


def _tc_mid(p, y, dis, b2, W):
  def body(p0_ref, p1_ref, y_ref, dis_ref, b_ref, w_ref, out_ref):
    d = dis_ref[...]
    t = jnp.maximum(
        d * (p0_ref[...] + p1_ref[...] - y_ref[...]) + b_ref[...], 0.0
    )
    out_ref[...] = d * jnp.dot(
        t, w_ref[...], preferred_element_type=jnp.float32
    )

  return pl.pallas_call(
      body,
      grid=(NP // BLK,),
      in_specs=[
          pl.BlockSpec((BLK, D), lambda i: (i, 0)),
          pl.BlockSpec((BLK, D), lambda i: (i, 0)),
          pl.BlockSpec((BLK, D), lambda i: (i, 0)),
          pl.BlockSpec((BLK, 1), lambda i: (i, 0)),
          pl.BlockSpec((1, D), lambda i: (0, 0)),
          pl.BlockSpec((D, D), lambda i: (0, 0)),
      ],
      out_specs=pl.BlockSpec((BLK, D), lambda i: (i, 0)),
      out_shape=jax.ShapeDtypeStruct((NP, D), jnp.float32),
  )(p[0], p[1], y, dis, b2, W)


def _tc_last(q, y, dis, b2):
  def body(q0_ref, q1_ref, y_ref, dis_ref, b_ref, out_ref):
    d = dis_ref[...]
    out_ref[...] = jnp.maximum(
        d * (q0_ref[...] + q1_ref[...] - y_ref[...]) + b_ref[...], 0.0
    )

  return pl.pallas_call(
      body,
      grid=(NP // BLK,),
      in_specs=[
          pl.BlockSpec((BLK, D), lambda i: (i, 0)),
          pl.BlockSpec((BLK, D), lambda i: (i, 0)),
          pl.BlockSpec((BLK, D), lambda i: (i, 0)),
          pl.BlockSpec((BLK, 1), lambda i: (i, 0)),
          pl.BlockSpec((1, D), lambda i: (0, 0)),
      ],
      out_specs=pl.BlockSpec((BLK, D), lambda i: (i, 0)),
      out_shape=jax.ShapeDtypeStruct((NP, D), jnp.float32),
  )(q[0], q[1], y, dis, b2)


def kernel(x, edge_index, W, b):
  E = edge_index.shape[1]
  row = edge_index[0].astype(jnp.int32)
  col = edge_index[1].astype(jnp.int32)

  epw = -(-E // NW)

  dchunk = -(-epw // DPH)
  dpad = dchunk * DPH * NW - E
  dpad_ar = jnp.arange(dpad, dtype=jnp.int32)
  col3 = jnp.concatenate([col, N + dpad_ar % (NP - N)]).reshape(
      NW, dchunk, DPH)

  nchunk = -(-epw // SCH)
  if nchunk % SPH:
    nchunk = (nchunk // SPH + 1) * SPH
  pad = nchunk * SCH * NW - E
  pad_ar = jnp.arange(pad, dtype=jnp.int32)
  row4 = jnp.concatenate([row, pad_ar % N]).reshape(
      NW, nchunk // SPH, SPH, SCH)
  col4 = jnp.concatenate([col, N + pad_ar % (NP - N)]).reshape(
      NW, nchunk // SPH, SPH, SCH)

  b2 = b.reshape(1, D)
  zeros_h = jnp.zeros((NP,), jnp.float32)
  ones_h = jnp.ones((DPH,), jnp.float32)
  xp = jnp.pad(x, ((0, NP - N), (0, 0)))

  h = _sc_degree(col3, zeros_h, ones_h, dchunk)
  y, dis = _tc_first(xp, W, h[0].reshape(NP, 1), h[1].reshape(NP, 1))
  p = _sc_scatter(y, row4, col4, nchunk)
  y2 = _tc_mid(p, y, dis, b2, W)
  q = _sc_scatter(y2, row4, col4, nchunk)
  return _tc_last(q, y2, dis, b2)[:N]
